# baseline (speedup 1.0000x reference)
import functools

import jax
import jax.numpy as jnp
from jax import lax
from jax.experimental import pallas as pl
from jax.experimental.pallas import tpu as pltpu
from jax.experimental.pallas import tpu_sc as plsc

_N = 16777216
_R = 1048576
_NC = 2
_NS = 16
_NW = _NC * _NS
_CHUNK = _N // _NW
_BUF = 32768
_STEPS = _CHUNK // _BUF
_HALF = _STEPS // 2
_R_WORKERS = _R // _CHUNK


@functools.partial(
    pl.kernel,
    out_type=jax.ShapeDtypeStruct((_N,), jnp.float32),
    mesh=plsc.VectorSubcoreMesh(core_axis_name="c", subcore_axis_name="s"),
    scratch_types=[
        pltpu.VMEM((2, _BUF), jnp.float32),
        pltpu.VMEM_SHARED((_NS, 2, _BUF), jnp.float32),
        pltpu.SemaphoreType.DMA,
        pltpu.SemaphoreType.DMA,
        pltpu.SemaphoreType.DMA,
        pltpu.SemaphoreType.DMA,
        pltpu.SemaphoreType.DMA,
        pltpu.SemaphoreType.DMA,
        pltpu.SemaphoreType.DMA,
        pltpu.SemaphoreType.DMA,
    ],
)
def _sc_assemble(fix_hbm, refi_hbm, out_hbm, tbuf, shared,
                 si0, si1, so0, so1, di0, di1, do0, do1):
    sid = lax.axis_index("s")
    wid = sid * _NC + lax.axis_index("c")
    base = wid * _CHUNK
    s_in, s_out = (si0, si1), (so0, so1)
    d_in, d_out = (di0, di1), (do0, do1)

    def _move(src_hbm, src_base):
        def src_at(g):
            return src_hbm.at[pl.ds(src_base + g * _BUF, _BUF)]

        def dst_at(g):
            return out_hbm.at[pl.ds(base + g * _BUF, _BUF)]

        def s_in_cp(t):
            return pltpu.make_async_copy(src_at(2 * t), tbuf.at[t % 2],
                                         s_in[t % 2])

        def s_out_cp(t):
            return pltpu.make_async_copy(tbuf.at[t % 2], dst_at(2 * t),
                                         s_out[t % 2])

        def d_in_cp(t):
            return pltpu.make_async_copy(src_at(2 * t + 1),
                                         shared.at[sid, t % 2], d_in[t % 2])

        def d_out_cp(t):
            return pltpu.make_async_copy(shared.at[sid, t % 2],
                                         dst_at(2 * t + 1), d_out[t % 2])

        s_in_cp(0).start()
        d_in_cp(0).start()
        for t in range(_HALF):
            if t + 1 < _HALF:
                if t >= 1:
                    s_out_cp(t - 1).wait()
                s_in_cp(t + 1).start()
                if t >= 1:
                    d_out_cp(t - 1).wait()
                d_in_cp(t + 1).start()
            s_in_cp(t).wait()
            s_out_cp(t).start()
            d_in_cp(t).wait()
            d_out_cp(t).start()
        for t in (_HALF - 2, _HALF - 1):
            s_out_cp(t).wait()
            d_out_cp(t).wait()

    @pl.when(wid < _R_WORKERS)
    def _():
        _move(refi_hbm, base)

    @pl.when(wid >= _R_WORKERS)
    def _():
        _move(fix_hbm, base)


def kernel(fixed_values, refinable_params, refinable_idx):
    del refinable_idx
    return _sc_assemble(fixed_values, refinable_params)
